# Initial kernel scaffold; baseline (speedup 1.0000x reference)
import jax
import jax.numpy as jnp
from jax.experimental import pallas as pl


def kernel(e, codebook):
    raise NotImplementedError("write your pallas kernel here")


# speedup vs baseline: 1.1650x; 1.1650x over previous
import jax
import jax.numpy as jnp
from jax import lax
from jax.experimental import pallas as pl
from jax.experimental.pallas import tpu as pltpu
from jax.experimental.pallas import tpu_sc as plsc

A = 8192
DIM = 256
N = 65536

BN = 256
BA = 512
T = A // BA
LW = 128
NC = BA // LW


WREAL = 2736
NWIN = 3
TW = 6
WPAD = TW * BA
APAD = NWIN * WPAD


def _argmin_body(e2_ref, en2_ref, cb_ref, cn2_ref, idx_ref):
    e2 = e2_ref[...]
    en2 = en2_ref[...]

    def window_argmin(h):
        best = [jnp.full((BN, LW), jnp.inf, dtype=jnp.float32)
                for _ in range(NC)]
        code = [jnp.zeros((BN, LW), dtype=jnp.int32) for _ in range(NC)]
        for ti in range(TW):
            t = h * TW + ti
            c_tile = cb_ref[t * BA:(t + 1) * BA, :]
            mm2 = lax.dot_general(
                e2, c_tile,
                (((1,), (1,)), ((), ())),
                preferred_element_type=jnp.float32,
            )
            for c in range(NC):
                cn2_t = cn2_ref[:, t * BA + c * LW: t * BA + (c + 1) * LW]
                dc = (en2 + mm2[:, c * LW:(c + 1) * LW]) + cn2_t
                upd = dc < best[c]
                best[c] = jnp.where(upd, dc, best[c])
                code[c] = jnp.where(upd, ti * NC + c, code[c])

        lane = lax.broadcasted_iota(jnp.int32, (BN, LW), 1)
        v, g = best[0], code[0] * LW + lane
        for c in range(1, NC):
            vc, gc = best[c], code[c] * LW + lane
            take = (vc < v) | ((vc == v) & (gc < g))
            v = jnp.where(take, vc, v)
            g = jnp.where(take, gc, g)
        tmin = jnp.min(v, axis=1, keepdims=True)
        fidx = jnp.min(jnp.where(v <= tmin, g, jnp.int32(APAD)),
                       axis=1, keepdims=True)
        return tmin, fidx + h * WREAL

    acc, idx = window_argmin(0)
    acc = acc.astype(jnp.bfloat16)
    for h in range(1, NWIN):
        m, g = window_argmin(h)
        accf = acc.astype(jnp.float32)
        use = m < accf
        idx = jnp.where(use, g, idx)
        acc = jnp.where(use, m, accf).astype(jnp.bfloat16)
    idx_ref[...] = idx


_argmin_call = pl.pallas_call(
    _argmin_body,
    grid=(N // BN,),
    in_specs=[
        pl.BlockSpec((BN, DIM), lambda i: (i, 0)),
        pl.BlockSpec((BN, 1), lambda i: (i, 0)),
        pl.BlockSpec((APAD, DIM), lambda i: (0, 0)),
        pl.BlockSpec((1, APAD), lambda i: (0, 0)),
    ],
    out_specs=pl.BlockSpec((BN, 1), lambda i: (i, 0)),
    out_shape=jax.ShapeDtypeStruct((N, 1), jnp.int32),
    compiler_params=pltpu.CompilerParams(
        dimension_semantics=("arbitrary",),
    ),
)


_SC_INFO = plsc.get_sparse_core_info()
_NW = _SC_INFO.num_cores * _SC_INFO.num_subcores
_BPW = N // _NW
_CH = 128
_NCH = _BPW // _CH


def _gather_body(cb_hbm, idx_hbm, out_hbm, idx_v, rows_v, sem):
    wid = lax.axis_index("s") * _SC_INFO.num_cores + lax.axis_index("c")
    base = wid * _BPW

    def chunk(k, _):
        off = base + k * _CH
        pltpu.sync_copy(idx_hbm.at[pl.ds(off, _CH)], idx_v)
        pltpu.async_copy(cb_hbm.at[idx_v], rows_v, sem).wait()
        pltpu.sync_copy(rows_v, out_hbm.at[pl.ds(off, _CH)])
        return 0

    lax.fori_loop(0, _NCH, chunk, 0)


_gather_call = pl.kernel(
    _gather_body,
    out_type=jax.ShapeDtypeStruct((N, DIM), jnp.float32),
    mesh=plsc.VectorSubcoreMesh(core_axis_name="c", subcore_axis_name="s"),
    scratch_types=[
        pltpu.VMEM((_CH,), jnp.int32),
        pltpu.VMEM((_CH, DIM), jnp.float32),
        pltpu.SemaphoreType.DMA,
    ],
)


def kernel(e, codebook):
    en2 = jnp.sum(e * e, axis=1, keepdims=True)
    cn2 = jnp.sum(codebook * codebook, axis=1)
    e2b = (e * -2.0).astype(jnp.bfloat16)
    cbb = codebook.astype(jnp.bfloat16)
    cbp = jnp.zeros((APAD, DIM), jnp.bfloat16)
    cn2p = jnp.full((1, APAD), jnp.inf, jnp.float32)
    for w in range(NWIN):
        lo = w * WREAL
        hi = min(lo + WREAL, A)
        cbp = cbp.at[w * WPAD:w * WPAD + (hi - lo)].set(cbb[lo:hi])
        cn2p = cn2p.at[:, w * WPAD:w * WPAD + (hi - lo)].set(cn2[None, lo:hi])
    idx = _argmin_call(e2b, en2, cbp, cn2p).reshape(N)
    q = _gather_call(codebook, idx)
    return (idx, q)
